# baseline (speedup 1.0000x reference)
import functools

import jax
import jax.numpy as jnp
from jax import lax
from jax.experimental import pallas as pl
from jax.experimental.pallas import tpu as pltpu
from jax.experimental.pallas import tpu_sc as plsc

_INFO = plsc.get_sparse_core_info()
_NC = _INFO.num_cores
_NS = _INFO.num_subcores
_NW = _NC * _NS
_NBUF = 4
_LANES = 128
_SUPER = 512


@functools.lru_cache(maxsize=None)
def _make_scan_extract(B, V, D):
    ncols = (V + _LANES - 1) // _LANES
    ntc = (ncols + _NW - 1) // _NW
    ngrp = (ntc + _NBUF - 1) // _NBUF
    cap = B + _SUPER
    dump = B
    mesh = plsc.VectorSubcoreMesh(core_axis_name="c", subcore_axis_name="s")

    @functools.partial(
        pl.kernel,
        mesh=mesh,
        out_type=[
            jax.ShapeDtypeStruct((_NW * cap, _LANES), jnp.float32),
            jax.ShapeDtypeStruct((_NW * cap,), jnp.int32),
            jax.ShapeDtypeStruct((_NW * 16,), jnp.int32),
        ],
        scratch_types=[
            pltpu.VMEM((B,), jnp.int32),
            pltpu.VMEM((B + 16,), jnp.int32),
            pltpu.VMEM((_NBUF, D, _LANES), jnp.float32),
            pltpu.VMEM((16,), jnp.int32),
            pltpu.VMEM((16,), jnp.int32),
            pltpu.VMEM((32, _LANES), jnp.float32),
            pltpu.VMEM((32,), jnp.int32),
            pltpu.VMEM((16,), jnp.int32),
            [pltpu.SemaphoreType.DMA] * _NBUF,
        ],
        compiler_params=pltpu.CompilerParams(
            use_tc_tiling_on_sc=True, needs_layout_passes=False
        ),
    )
    def scan_extract(idx_hbm, table_hbm, stage_hbm, pos_hbm, cnt_hbm,
                     idx_v, lr_v, blocks_v, tb_i, tb_r, ab_v, abp_v,
                     cnt_v, sems):
        wid = lax.axis_index("s") * _NC + lax.axis_index("c")
        c_lo = wid * ntc
        base1 = wid * cap
        iota16 = lax.iota(jnp.int32, 16)
        lane0 = iota16 == 0
        zeros16 = jnp.zeros((16,), jnp.int32)
        pltpu.sync_copy(idx_hbm, idx_v)

        def bin_body(t, cnt):
            v = idx_v[pl.ds(t * 16, 16)]
            c = lax.shift_right_logical(v, 7)
            m = (c >= c_lo) & (c < c_lo + ntc)
            plsc.store_compressed(
                lr_v.at[pl.ds(cnt, 16)], t * 16 + iota16, mask=m
            )
            return cnt + plsc.all_reduce_population_count(m)[0]

        cnt = lax.fori_loop(0, B // 16, bin_body, 0)
        nq = lax.shift_right_logical(cnt + 63, 6)
        cnt_vec = jnp.full((16,), cnt, jnp.int32)

        def fetch(b, ci):
            valid = (ci < ntc) & (c_lo + ci < ncols)

            @pl.when(valid)
            def _():
                off = pl.multiple_of((c_lo + ci) * _LANES, _LANES)
                pltpu.async_copy(
                    table_hbm.at[:, pl.ds(off, _LANES)],
                    blocks_v.at[b],
                    sems[b],
                )

        def append_row(b, k, ab_cnt):
            i_sp = plsc.load_gather(tb_i, [jnp.full((16,), k, jnp.int32)])
            r_sp = plsc.load_gather(tb_r, [jnp.full((16,), k, jnp.int32)])
            lane_vec = i_sp & (_LANES - 1)
            slot_vec = jnp.full((16,), ab_cnt & 31, jnp.int32)
            for jj in range(D // 16):
                vals = plsc.load_gather(
                    blocks_v.at[b], [jj * 16 + iota16, lane_vec]
                )
                plsc.store_scatter(ab_v, [slot_vec, jj * 16 + iota16], vals)
            plsc.store_scatter(abp_v, [slot_vec], r_sp, mask=lane0)

            @pl.when((ab_cnt & 7) == 7)
            def _():
                w0 = pl.multiple_of(ab_cnt & 24, 8)
                g0 = pl.multiple_of(base1 + (ab_cnt & ~7), 8)
                pltpu.sync_copy(ab_v.at[pl.ds(w0, 8)],
                                stage_hbm.at[pl.ds(g0, 8)])
                pltpu.sync_copy(abp_v.at[pl.ds(w0, 8)],
                                pos_hbm.at[pl.ds(g0, 8)])

            return ab_cnt + 1

        for b in range(_NBUF):
            fetch(b, b)

        def grp_body(g, ab_cnt):
            for b in range(_NBUF):
                ci = g * _NBUF + b
                valid = (ci < ntc) & (c_lo + ci < ncols)

                @pl.when(valid)
                def _():
                    pltpu.make_async_copy(
                        table_hbm.at[:, pl.ds(0, _LANES)],
                        blocks_v.at[b],
                        sems[b],
                    ).wait()

                col = c_lo + ci

                def rescan4(q, ab_cnt):
                    rvs, ivs, m2s, nms = [], [], [], []
                    for u in range(4):
                        t2 = q * 4 + u
                        ent = (t2 * 16 + iota16) < cnt_vec
                        rv_raw = lr_v[pl.ds(t2 * 16, 16)]
                        rv = jnp.where(ent, rv_raw, zeros16)
                        iv = plsc.load_gather(idx_v, [rv])
                        m2 = (lax.shift_right_logical(iv, 7) == col) & ent
                        rvs.append(rv)
                        ivs.append(iv)
                        m2s.append(m2)
                        nms.append(plsc.all_reduce_population_count(m2)[0])

                    def matches(ab_cnt):
                        for u in range(4):
                            def one(u=u):
                                plsc.store_compressed(
                                    tb_i.at[pl.ds(0, 16)], ivs[u],
                                    mask=m2s[u])
                                plsc.store_compressed(
                                    tb_r.at[pl.ds(0, 16)], rvs[u],
                                    mask=m2s[u])

                            ab_cnt = lax.cond(
                                nms[u] > 0,
                                lambda a, one=one, u=u: (
                                    one() or lax.fori_loop(
                                        0, nms[u],
                                        lambda k, aa: append_row(b, k, aa),
                                        a)),
                                lambda a: a,
                                ab_cnt,
                            )
                        return ab_cnt

                    return lax.cond(
                        nms[0] + nms[1] + nms[2] + nms[3] > 0,
                        matches, lambda a: a, ab_cnt)

                ab_cnt = lax.cond(
                    valid,
                    lambda a: lax.fori_loop(0, nq, rescan4, a),
                    lambda a: a,
                    ab_cnt,
                )
                fetch(b, ci + _NBUF)
            return ab_cnt

        ab_cnt = lax.fori_loop(0, ngrp, grp_body, 0)

        def pad_body(_, a):
            slot_vec = jnp.full((16,), a & 31, jnp.int32)
            plsc.store_scatter(abp_v, [slot_vec],
                               jnp.full((16,), dump, jnp.int32), mask=lane0)

            @pl.when((a & 7) == 7)
            def _():
                w0 = pl.multiple_of(a & 24, 8)
                g0 = pl.multiple_of(base1 + (a & ~7), 8)
                pltpu.sync_copy(ab_v.at[pl.ds(w0, 8)],
                                stage_hbm.at[pl.ds(g0, 8)])
                pltpu.sync_copy(abp_v.at[pl.ds(w0, 8)],
                                pos_hbm.at[pl.ds(g0, 8)])

            return a + 1

        npad = (-ab_cnt) & (_SUPER - 1)
        total = lax.fori_loop(0, npad, pad_body, ab_cnt)

        cnt_v[...] = jnp.full((16,), total, jnp.int32)
        pltpu.sync_copy(cnt_v, cnt_hbm.at[pl.ds(wid * 16, 16)])

    return scan_extract, cap


@functools.lru_cache(maxsize=None)
def _make_scatter(B, cap):
    nsc = _SUPER // _LANES
    mesh = plsc.VectorSubcoreMesh(core_axis_name="c", subcore_axis_name="s")

    @functools.partial(
        pl.kernel,
        mesh=mesh,
        out_type=jax.ShapeDtypeStruct((B + 8, _LANES), jnp.float32),
        scratch_types=[
            pltpu.VMEM((_SUPER, _LANES), jnp.float32),
            pltpu.VMEM((nsc, _LANES), jnp.int32),
            pltpu.VMEM((16,), jnp.int32),
            pltpu.SemaphoreType.DMA,
        ],
        compiler_params=pltpu.CompilerParams(use_tc_tiling_on_sc=False),
    )
    def scatter(stage_hbm, pos_hbm, cnt_hbm, out_hbm, rows_v, pos_v, cnt_v,
                sem):
        wid = lax.axis_index("s") * _NC + lax.axis_index("c")
        base1 = wid * cap
        pltpu.sync_copy(cnt_hbm.at[pl.ds(wid * 16, 16)], cnt_v)
        n2 = cnt_v[...][0]

        def super_chunk(s, carry):
            off = base1 + s * _SUPER
            pltpu.sync_copy(stage_hbm.at[pl.ds(off, _SUPER)], rows_v)
            for j in range(nsc):
                pltpu.sync_copy(
                    pos_hbm.at[pl.ds(off + j * _LANES, _LANES)], pos_v.at[j]
                )
            for j in range(nsc):
                pltpu.async_copy(
                    rows_v.at[pl.ds(j * _LANES, _LANES)],
                    out_hbm.at[pos_v.at[j]],
                    sem,
                )
            for j in range(nsc):
                pltpu.make_async_copy(
                    rows_v.at[pl.ds(j * _LANES, _LANES)],
                    out_hbm.at[pl.ds(0, _LANES)],
                    sem,
                ).wait()
            return carry

        lax.fori_loop(0, lax.shift_right_logical(n2, 9), super_chunk, 0)

    return scatter


def kernel(x, weight):
    (B,) = x.shape
    V, D = weight.shape
    assert B % (16 * _NW) == 0
    idx = x.astype(jnp.int32)
    scan_extract, cap = _make_scan_extract(B, V, D)
    stage, pos, cnts = scan_extract(idx, weight.T)
    out_pad = _make_scatter(B, cap)(stage, pos, cnts)
    return out_pad[:B, :D]


# speedup vs baseline: 1.0920x; 1.0920x over previous
import functools

import jax
import jax.numpy as jnp
from jax import lax
from jax.experimental import pallas as pl
from jax.experimental.pallas import tpu as pltpu
from jax.experimental.pallas import tpu_sc as plsc

_INFO = plsc.get_sparse_core_info()
_NC = _INFO.num_cores
_NS = _INFO.num_subcores
_NW = _NC * _NS
_NBUF = 2
_LANES = 128
_SUPER = 512
_NSUB = 4
_SUBW = 64


@functools.lru_cache(maxsize=None)
def _make_scan_extract(B, V, D):
    ncols = (V + _LANES - 1) // _LANES
    ntc = (ncols + _NW - 1) // _NW
    cap = B + _SUPER
    dump = B
    assert ntc <= _NSUB * _SUBW
    mesh = plsc.VectorSubcoreMesh(core_axis_name="c", subcore_axis_name="s")

    @functools.partial(
        pl.kernel,
        mesh=mesh,
        out_type=[
            jax.ShapeDtypeStruct((_NW * cap, _LANES), jnp.float32),
            jax.ShapeDtypeStruct((_NW * cap,), jnp.int32),
            jax.ShapeDtypeStruct((_NW * 16,), jnp.int32),
        ],
        scratch_types=[
            pltpu.VMEM((B,), jnp.int32),
            [pltpu.VMEM((B + 16,), jnp.int32)] * _NSUB,
            pltpu.VMEM((_NBUF, D, _LANES), jnp.float32),
            pltpu.VMEM((16,), jnp.int32),
            pltpu.VMEM((16,), jnp.int32),
            pltpu.VMEM((32, _LANES), jnp.float32),
            pltpu.VMEM((32,), jnp.int32),
            pltpu.VMEM((16,), jnp.int32),
            [pltpu.SemaphoreType.DMA] * _NBUF,
        ],
        compiler_params=pltpu.CompilerParams(
            use_tc_tiling_on_sc=True, needs_layout_passes=False
        ),
    )
    def scan_extract(idx_hbm, table_hbm, stage_hbm, pos_hbm, cnt_hbm,
                     idx_v, lrs, blocks_v, tb_i, tb_r, ab_v, abp_v,
                     cnt_v, sems):
        wid = lax.axis_index("s") * _NC + lax.axis_index("c")
        c_lo = wid * ntc
        base1 = wid * cap
        iota16 = lax.iota(jnp.int32, 16)
        lane0 = iota16 == 0
        pltpu.sync_copy(idx_hbm, idx_v)

        def bin_body(t, cnts):
            v = idx_v[pl.ds(t * 16, 16)]
            c_rel = lax.shift_right_logical(v, 7) - c_lo
            m = (c_rel >= 0) & (c_rel < ntc)
            sub = lax.shift_right_logical(c_rel, 6)
            out = []
            for u in range(_NSUB):
                ms = m & (sub == u)
                plsc.store_compressed(
                    lrs[u].at[pl.ds(cnts[u], 16)], t * 16 + iota16, mask=ms
                )
                out.append(
                    cnts[u] + plsc.all_reduce_population_count(ms)[0]
                )
            return tuple(out)

        cnts = lax.fori_loop(0, B // 16, bin_body, (0,) * _NSUB)
        cnt_vecs = [jnp.full((16,), c, jnp.int32) for c in cnts]
        nchs = [lax.shift_right_logical(c + 15, 4) for c in cnts]

        def fetch(b, ci):
            valid = (ci < ntc) & (c_lo + ci < ncols)

            @pl.when(valid)
            def _():
                off = pl.multiple_of((c_lo + ci) * _LANES, _LANES)
                pltpu.async_copy(
                    table_hbm.at[:, pl.ds(off, _LANES)],
                    blocks_v.at[b],
                    sems[b],
                )

        def append_row(b, k, ab_cnt):
            i_sp = plsc.load_gather(tb_i, [jnp.full((16,), k, jnp.int32)])
            r_sp = plsc.load_gather(tb_r, [jnp.full((16,), k, jnp.int32)])
            lane_vec = i_sp & (_LANES - 1)
            slot_vec = jnp.full((16,), ab_cnt & 31, jnp.int32)
            for jj in range(D // 16):
                vals = plsc.load_gather(
                    blocks_v.at[b], [jj * 16 + iota16, lane_vec]
                )
                plsc.store_scatter(ab_v, [slot_vec, jj * 16 + iota16], vals)
            plsc.store_scatter(abp_v, [slot_vec], r_sp, mask=lane0)

            @pl.when((ab_cnt & 7) == 7)
            def _():
                w0 = pl.multiple_of(ab_cnt & 24, 8)
                g0 = pl.multiple_of(base1 + (ab_cnt & ~7), 8)
                pltpu.sync_copy(ab_v.at[pl.ds(w0, 8)],
                                stage_hbm.at[pl.ds(g0, 8)])
                pltpu.sync_copy(abp_v.at[pl.ds(w0, 8)],
                                pos_hbm.at[pl.ds(g0, 8)])

            return ab_cnt + 1

        ab_cnt = 0
        for u in range(_NSUB):
            if u == 0:
                for b in range(_NBUF):
                    fetch(b, b)

            def grp_body(g, ab_cnt, u=u):
                for b in range(_NBUF):
                    ci = u * _SUBW + g * _NBUF + b
                    valid = (ci < ntc) & (c_lo + ci < ncols)

                    @pl.when(valid)
                    def _():
                        pltpu.make_async_copy(
                            table_hbm.at[:, pl.ds(0, _LANES)],
                            blocks_v.at[b],
                            sems[b],
                        ).wait()

                    col = c_lo + ci

                    def rescan(t2, ab_cnt, u=u, b=b, col=col):
                        rv = lrs[u][pl.ds(t2 * 16, 16)] & (B - 1)
                        iv = plsc.load_gather(idx_v, [rv])
                        ent = (t2 * 16 + iota16) < cnt_vecs[u]
                        m2 = (lax.shift_right_logical(iv, 7) == col) & ent
                        nm = plsc.all_reduce_population_count(m2)[0]

                        def matches(ab_cnt):
                            plsc.store_compressed(tb_i.at[pl.ds(0, 16)], iv,
                                                  mask=m2)
                            plsc.store_compressed(tb_r.at[pl.ds(0, 16)], rv,
                                                  mask=m2)
                            return lax.fori_loop(
                                0, nm,
                                lambda k, a: append_row(b, k, a),
                                ab_cnt,
                            )

                        return lax.cond(nm > 0, matches, lambda a: a, ab_cnt)

                    ab_cnt = lax.cond(
                        valid,
                        lambda a, u=u: lax.fori_loop(0, nchs[u], rescan, a),
                        lambda a: a,
                        ab_cnt,
                    )
                    fetch(b, ci + _NBUF)
                return ab_cnt

            ab_cnt = lax.fori_loop(0, _SUBW // _NBUF, grp_body, ab_cnt)

        def pad_body(_, a):
            slot_vec = jnp.full((16,), a & 31, jnp.int32)
            plsc.store_scatter(abp_v, [slot_vec],
                               jnp.full((16,), dump, jnp.int32), mask=lane0)

            @pl.when((a & 7) == 7)
            def _():
                w0 = pl.multiple_of(a & 24, 8)
                g0 = pl.multiple_of(base1 + (a & ~7), 8)
                pltpu.sync_copy(ab_v.at[pl.ds(w0, 8)],
                                stage_hbm.at[pl.ds(g0, 8)])
                pltpu.sync_copy(abp_v.at[pl.ds(w0, 8)],
                                pos_hbm.at[pl.ds(g0, 8)])

            return a + 1

        npad = (-ab_cnt) & (_SUPER - 1)
        total = lax.fori_loop(0, npad, pad_body, ab_cnt)

        cnt_v[...] = jnp.full((16,), total, jnp.int32)
        pltpu.sync_copy(cnt_v, cnt_hbm.at[pl.ds(wid * 16, 16)])

    return scan_extract, cap


@functools.lru_cache(maxsize=None)
def _make_scatter(B, cap):
    nsc = _SUPER // _LANES
    mesh = plsc.VectorSubcoreMesh(core_axis_name="c", subcore_axis_name="s")

    @functools.partial(
        pl.kernel,
        mesh=mesh,
        out_type=jax.ShapeDtypeStruct((B + 8, _LANES), jnp.float32),
        scratch_types=[
            pltpu.VMEM((_SUPER, _LANES), jnp.float32),
            pltpu.VMEM((nsc, _LANES), jnp.int32),
            pltpu.VMEM((16,), jnp.int32),
            pltpu.SemaphoreType.DMA,
        ],
        compiler_params=pltpu.CompilerParams(use_tc_tiling_on_sc=False),
    )
    def scatter(stage_hbm, pos_hbm, cnt_hbm, out_hbm, rows_v, pos_v, cnt_v,
                sem):
        wid = lax.axis_index("s") * _NC + lax.axis_index("c")
        base1 = wid * cap
        pltpu.sync_copy(cnt_hbm.at[pl.ds(wid * 16, 16)], cnt_v)
        n2 = cnt_v[...][0]

        def super_chunk(s, carry):
            off = base1 + s * _SUPER
            pltpu.sync_copy(stage_hbm.at[pl.ds(off, _SUPER)], rows_v)
            for j in range(nsc):
                pltpu.sync_copy(
                    pos_hbm.at[pl.ds(off + j * _LANES, _LANES)], pos_v.at[j]
                )
            for j in range(nsc):
                pltpu.async_copy(
                    rows_v.at[pl.ds(j * _LANES, _LANES)],
                    out_hbm.at[pos_v.at[j]],
                    sem,
                )
            for j in range(nsc):
                pltpu.make_async_copy(
                    rows_v.at[pl.ds(j * _LANES, _LANES)],
                    out_hbm.at[pl.ds(0, _LANES)],
                    sem,
                ).wait()
            return carry

        lax.fori_loop(0, lax.shift_right_logical(n2, 9), super_chunk, 0)

    return scatter


def kernel(x, weight):
    (B,) = x.shape
    V, D = weight.shape
    assert B % (16 * _NW) == 0 and (B & (B - 1)) == 0
    idx = x.astype(jnp.int32)
    scan_extract, cap = _make_scan_extract(B, V, D)
    stage, pos, cnts = scan_extract(idx, weight.T)
    out_pad = _make_scatter(B, cap)(stage, pos, cnts)
    return out_pad[:B, :D]


# speedup vs baseline: 2.8955x; 2.6515x over previous
import functools

import jax
import jax.numpy as jnp
from jax import lax
from jax.experimental import pallas as pl
from jax.experimental.pallas import tpu as pltpu
from jax.experimental.pallas import tpu_sc as plsc

_INFO = plsc.get_sparse_core_info()
_NC = _INFO.num_cores
_NS = _INFO.num_subcores
_NW = _NC * _NS
_NBUF = 8
_LANES = 128


@functools.lru_cache(maxsize=None)
def _make_embed(B, V, D):
    KC = B // _NW
    mesh = plsc.VectorSubcoreMesh(core_axis_name="c", subcore_axis_name="s")

    @functools.partial(
        pl.kernel,
        mesh=mesh,
        out_type=jax.ShapeDtypeStruct((D, B), jnp.float32),
        scratch_types=[
            pltpu.VMEM((KC,), jnp.int32),
            pltpu.VMEM((_NBUF, D, _LANES), jnp.float32),
            pltpu.VMEM((D, KC), jnp.float32),
            [pltpu.SemaphoreType.DMA] * _NBUF,
        ],
        compiler_params=pltpu.CompilerParams(
            use_tc_tiling_on_sc=True, needs_layout_passes=False
        ),
    )
    def embed(idx_hbm, table_hbm, out_hbm, idx_v, blocks_v, cols_v, sems):
        wid = lax.axis_index("s") * _NC + lax.axis_index("c")
        base = wid * KC
        pltpu.sync_copy(idx_hbm.at[pl.ds(base, KC)], idx_v)
        iota16 = lax.iota(jnp.int32, 16)

        def splat(e):
            return plsc.load_gather(idx_v, [jnp.full((16,), e, jnp.int32)])

        def fetch(b, e):
            off = pl.multiple_of((splat(e) & -_LANES)[0], _LANES)
            pltpu.async_copy(
                table_hbm.at[:, pl.ds(off, _LANES)], blocks_v.at[b], sems[b]
            )

        def extract(b, e):
            lane_vec = splat(e) & (_LANES - 1)
            e_vec = jnp.full((16,), e, jnp.int32)
            for jj in range(D // 16):
                vals = plsc.load_gather(
                    blocks_v.at[b], [jj * 16 + iota16, lane_vec]
                )
                plsc.store_scatter(cols_v, [jj * 16 + iota16, e_vec], vals)

        for b in range(_NBUF):
            fetch(b, b)

        def body(g, carry):
            for b in range(_NBUF):
                e = g * _NBUF + b
                pltpu.make_async_copy(
                    table_hbm.at[:, pl.ds(0, _LANES)], blocks_v.at[b], sems[b]
                ).wait()
                extract(b, e)
                nxt = e + _NBUF

                @pl.when(nxt < KC)
                def _():
                    fetch(b, nxt)

            return carry

        lax.fori_loop(0, KC // _NBUF, body, 0)
        pltpu.sync_copy(cols_v, out_hbm.at[:, pl.ds(base, KC)])

    return embed


def kernel(x, weight):
    (B,) = x.shape
    V, D = weight.shape
    assert B % (_NW * _NBUF) == 0
    idx = x.astype(jnp.int32)
    out_t = _make_embed(B, V, D)(idx, weight.T)
    return out_t.T
